# Initial kernel scaffold; baseline (speedup 1.0000x reference)
import functools

import jax
import jax.numpy as jnp
from jax import lax
from jax.experimental import pallas as pl
from jax.experimental.pallas import tpu as pltpu
from jax.experimental.pallas import tpu_sc as plsc

N_NODES = 50000
N_EDGES = 800000
D = 64
LAYERS = 3

NC = 2
NS = 16
N_PAD = 50048
ROWS_SC = N_PAD // NC
ROWS_TILE = ROWS_SC // NS
DUMMY = ROWS_SC
ACC_ROWS = ROWS_SC + 8
K = 128
E_TILE_BLOCKS = 391
E_TILE = E_TILE_BLOCKS * K
E_PAD = E_TILE * NS
OUT_OF_RANGE = N_PAD + 512

_mesh = plsc.VectorSubcoreMesh(core_axis_name="c", subcore_axis_name="s")


@functools.partial(
    pl.kernel,
    mesh=_mesh,
    out_type=jax.ShapeDtypeStruct((N_PAD, D), jnp.float32),
    scratch_types=[
        pltpu.VMEM((K,), jnp.int32),
        pltpu.VMEM((K,), jnp.int32),
        pltpu.VMEM((K,), jnp.int32),
        pltpu.VMEM((K,), jnp.float32),
        pltpu.VMEM((K, D), jnp.float32),
        pltpu.VMEM_SHARED((ACC_ROWS, D), jnp.float32),
        pltpu.SemaphoreType.DMA,
    ],
)
def _layer(x_hbm, src_hbm, dst_hbm, w_hbm, out_hbm,
           src_v, dst_v, ldst_v, w_v, rows_v, acc, sem):
    c = lax.axis_index("c")
    s = lax.axis_index("s")

    zero = jnp.zeros((16,), jnp.float32)
    for r in range(K):
        for d4 in range(D // 16):
            rows_v[r, pl.ds(d4 * 16, 16)] = zero
    base = s * ROWS_TILE
    n_full = ROWS_TILE // K
    rem = ROWS_TILE % K
    for j in range(n_full):
        pltpu.sync_copy(rows_v, acc.at[pl.ds(base + j * K, K)])
    pltpu.sync_copy(rows_v.at[pl.ds(0, rem)],
                    acc.at[pl.ds(base + n_full * K, rem)])

    @pl.when(s == 0)
    def _zero_dummy():
        pltpu.sync_copy(rows_v.at[pl.ds(0, ACC_ROWS - ROWS_SC)],
                        acc.at[pl.ds(ROWS_SC, ACC_ROWS - ROWS_SC)])

    plsc.subcore_barrier()

    ebase = s * E_TILE
    row_off = c * ROWS_SC

    def body(b, carry):
        eb = ebase + b * K
        pltpu.sync_copy(src_hbm.at[pl.ds(eb, K)], src_v)
        pltpu.sync_copy(dst_hbm.at[pl.ds(eb, K)], dst_v)
        pltpu.sync_copy(w_hbm.at[pl.ds(eb, K)], w_v)
        pltpu.async_copy(x_hbm.at[src_v], rows_v, sem).wait()
        for g in range(K // 16):
            d16 = dst_v[pl.ds(g * 16, 16)]
            loc = d16 - row_off
            ok = (loc >= 0) & (loc < ROWS_SC)
            ldst_v[pl.ds(g * 16, 16)] = jnp.where(ok, loc, DUMMY)
            for e in range(16):
                idx = g * 16 + e
                ws = w_v[idx]
                for d4 in range(D // 16):
                    sl = pl.ds(d4 * 16, 16)
                    rows_v[idx, sl] = rows_v[idx, sl] * ws
        pltpu.sync_copy(rows_v, acc.at[ldst_v], add=True)
        return carry

    lax.fori_loop(0, E_TILE_BLOCKS, body, 0)

    plsc.subcore_barrier()

    for j in range(n_full):
        pltpu.sync_copy(acc.at[pl.ds(base + j * K, K)], rows_v)
        pltpu.sync_copy(rows_v, out_hbm.at[pl.ds(row_off + base + j * K, K)])
    pltpu.sync_copy(acc.at[pl.ds(base + n_full * K, rem)],
                    rows_v.at[pl.ds(0, rem)])
    pltpu.sync_copy(rows_v.at[pl.ds(0, rem)],
                    out_hbm.at[pl.ds(row_off + base + n_full * K, rem)])


def _mean_body(a_ref, b_ref, c_ref, d_ref, o_ref):
    o_ref[...] = (a_ref[...] + b_ref[...] + c_ref[...] + d_ref[...]) * 0.25


_BLK = 1000
_mean4 = pl.pallas_call(
    _mean_body,
    out_shape=jax.ShapeDtypeStruct((N_NODES, D), jnp.float32),
    grid=(N_NODES // _BLK,),
    in_specs=[pl.BlockSpec((_BLK, D), lambda i: (i, 0))] * 4,
    out_specs=pl.BlockSpec((_BLK, D), lambda i: (i, 0)),
)


def kernel(ItemAndUserEmebddings, edge_index, edge_weight):
    x = ItemAndUserEmebddings
    src = edge_index[1].astype(jnp.int32)
    dst = edge_index[0].astype(jnp.int32)
    pad = E_PAD - N_EDGES
    src_p = jnp.concatenate([src, jnp.zeros((pad,), jnp.int32)])
    dst_p = jnp.concatenate([dst, jnp.full((pad,), OUT_OF_RANGE, jnp.int32)])
    w_p = jnp.concatenate([edge_weight, jnp.zeros((pad,), jnp.float32)])
    x_p = jnp.concatenate(
        [x, jnp.zeros((N_PAD - N_NODES, D), jnp.float32)], axis=0)

    outs = [x]
    h = x_p
    for _ in range(LAYERS):
        h = _layer(h, src_p, dst_p, w_p)
        outs.append(h[:N_NODES])

    stacked = jnp.stack(outs, axis=1)
    mean = _mean4(outs[0], outs[1], outs[2], outs[3])
    return (mean, stacked)


# speedup vs baseline: 2.9684x; 2.9684x over previous
import functools

import jax
import jax.numpy as jnp
from jax import lax
from jax.experimental import pallas as pl
from jax.experimental.pallas import tpu as pltpu
from jax.experimental.pallas import tpu_sc as plsc

N_NODES = 50000
N_EDGES = 800000
D = 64
LAYERS = 3

NC = 2
NS = 16
N_PAD = 50176
ROWS_SC = N_PAD // NC
ROWS_TILE = ROWS_SC // NS
DUMMY = ROWS_SC
ACC_ROWS = ROWS_SC + 8
K = 128
E_TILE_BLOCKS = 391
E_TILE = E_TILE_BLOCKS * K
E_PAD = E_TILE * NS
OUT_OF_RANGE = N_PAD + 512

_mesh = plsc.VectorSubcoreMesh(core_axis_name="c", subcore_axis_name="s")


@functools.partial(
    pl.kernel,
    mesh=_mesh,
    compiler_params=pltpu.CompilerParams(use_tc_tiling_on_sc=False),
    out_type=jax.ShapeDtypeStruct((N_PAD, D), jnp.float32),
    scratch_types=[
        pltpu.VMEM((K,), jnp.int32),
        pltpu.VMEM((K,), jnp.int32),
        pltpu.VMEM((K,), jnp.int32),
        pltpu.VMEM((K,), jnp.float32),
        pltpu.VMEM((K, D), jnp.float32),
        pltpu.VMEM_SHARED((ACC_ROWS, D), jnp.float32),
        pltpu.SemaphoreType.DMA,
    ],
)
def _layer(x_hbm, src_hbm, dst_hbm, w_hbm, out_hbm,
           src_v, dst_v, ldst_v, w_v, rows_v, acc, sem):
    c = lax.axis_index("c")
    s = lax.axis_index("s")

    zero = jnp.zeros((16,), jnp.float32)
    for r in range(K):
        for d4 in range(D // 16):
            rows_v[r, pl.ds(d4 * 16, 16)] = zero
    base = s * ROWS_TILE
    n_full = ROWS_TILE // K
    rem = ROWS_TILE % K
    for j in range(n_full):
        pltpu.sync_copy(rows_v, acc.at[pl.ds(base + j * K, K)])
    pltpu.sync_copy(rows_v.at[pl.ds(0, rem)],
                    acc.at[pl.ds(base + n_full * K, rem)])

    @pl.when(s == 0)
    def _zero_dummy():
        pltpu.sync_copy(rows_v.at[pl.ds(0, ACC_ROWS - ROWS_SC)],
                        acc.at[pl.ds(ROWS_SC, ACC_ROWS - ROWS_SC)])

    plsc.subcore_barrier()

    ebase = s * E_TILE
    row_off = c * ROWS_SC

    def body(b, carry):
        eb = ebase + b * K
        pltpu.sync_copy(src_hbm.at[pl.ds(eb, K)], src_v)
        pltpu.sync_copy(dst_hbm.at[pl.ds(eb, K)], dst_v)
        pltpu.sync_copy(w_hbm.at[pl.ds(eb, K)], w_v)
        pltpu.async_copy(x_hbm.at[src_v], rows_v, sem).wait()
        for g in range(K // 16):
            d16 = dst_v[pl.ds(g * 16, 16)]
            loc = d16 - row_off
            ok = (loc >= 0) & (loc < ROWS_SC)
            ldst_v[pl.ds(g * 16, 16)] = jnp.where(ok, loc, DUMMY)
            w16 = w_v[pl.ds(g * 16, 16)]
            for e in range(16):
                idx = g * 16 + e
                ws = w16[e]
                for d4 in range(D // 16):
                    sl = pl.ds(d4 * 16, 16)
                    rows_v[idx, sl] = rows_v[idx, sl] * ws
        pltpu.sync_copy(rows_v, acc.at[ldst_v], add=True)
        return carry

    lax.fori_loop(0, E_TILE_BLOCKS, body, 0)

    plsc.subcore_barrier()

    for j in range(n_full):
        pltpu.sync_copy(acc.at[pl.ds(base + j * K, K)], rows_v)
        pltpu.sync_copy(rows_v, out_hbm.at[pl.ds(row_off + base + j * K, K)])
    pltpu.sync_copy(acc.at[pl.ds(base + n_full * K, rem)],
                    rows_v.at[pl.ds(0, rem)])
    pltpu.sync_copy(rows_v.at[pl.ds(0, rem)],
                    out_hbm.at[pl.ds(row_off + base + n_full * K, rem)])


def _mean_body(a_ref, b_ref, c_ref, d_ref, o_ref):
    o_ref[...] = (a_ref[...] + b_ref[...] + c_ref[...] + d_ref[...]) * 0.25


_BLK = 1000
_mean4 = pl.pallas_call(
    _mean_body,
    out_shape=jax.ShapeDtypeStruct((N_NODES, D), jnp.float32),
    grid=(N_NODES // _BLK,),
    in_specs=[pl.BlockSpec((_BLK, D), lambda i: (i, 0))] * 4,
    out_specs=pl.BlockSpec((_BLK, D), lambda i: (i, 0)),
)


def kernel(ItemAndUserEmebddings, edge_index, edge_weight):
    x = ItemAndUserEmebddings
    src = edge_index[1].astype(jnp.int32)
    dst = edge_index[0].astype(jnp.int32)
    pad = E_PAD - N_EDGES
    src_p = jnp.concatenate([src, jnp.zeros((pad,), jnp.int32)])
    dst_p = jnp.concatenate([dst, jnp.full((pad,), OUT_OF_RANGE, jnp.int32)])
    w_p = jnp.concatenate([edge_weight, jnp.zeros((pad,), jnp.float32)])
    x_p = jnp.concatenate(
        [x, jnp.zeros((N_PAD - N_NODES, D), jnp.float32)], axis=0)

    outs = [x]
    h = x_p
    for _ in range(LAYERS):
        h = _layer(h, src_p, dst_p, w_p)
        outs.append(h[:N_NODES])

    stacked = jnp.stack(outs, axis=1)
    mean = _mean4(outs[0], outs[1], outs[2], outs[3])
    return (mean, stacked)


# speedup vs baseline: 4.7085x; 1.5862x over previous
import functools

import jax
import jax.numpy as jnp
from jax import lax
from jax.experimental import pallas as pl
from jax.experimental.pallas import tpu as pltpu
from jax.experimental.pallas import tpu_sc as plsc

N_NODES = 50000
N_EDGES = 800000
D = 64
LAYERS = 3

NC = 2
NS = 16
N_PAD = 50176
ROWS_SC = N_PAD // NC
ROWS_TILE = ROWS_SC // NS
DUMMY = ROWS_SC
ACC_ROWS = ROWS_SC + 8
K = 128
BLOCKS = 393
E_TILE = BLOCKS * K
E_PAD = E_TILE * NS
OUT_OF_RANGE = N_PAD + 512
NBUF = 3

_mesh = plsc.VectorSubcoreMesh(core_axis_name="c", subcore_axis_name="s")


@functools.partial(
    pl.kernel,
    mesh=_mesh,
    compiler_params=pltpu.CompilerParams(use_tc_tiling_on_sc=False),
    out_type=jax.ShapeDtypeStruct((N_PAD, D), jnp.float32),
    scratch_types=[
        pltpu.VMEM((NBUF, 2, K), jnp.int32),
        pltpu.VMEM((NBUF, K), jnp.float32),
        pltpu.VMEM((NBUF, K), jnp.int32),
        pltpu.VMEM((NBUF, K, D), jnp.float32),
        pltpu.VMEM_SHARED((ACC_ROWS, D), jnp.float32),
        pltpu.SemaphoreType.DMA((NBUF,)),
        pltpu.SemaphoreType.DMA((NBUF,)),
        pltpu.SemaphoreType.DMA((NBUF,)),
    ],
)
def _layer(x_hbm, packed_hbm, w_hbm, out_hbm,
           idxw_v, w_v, ldst_v, rows_v, acc, isem, gsem, ssem):
    c = lax.axis_index("c")
    s = lax.axis_index("s")

    zero = jnp.zeros((16,), jnp.float32)
    for r in range(K):
        for d4 in range(D // 16):
            rows_v[0, r, pl.ds(d4 * 16, 16)] = zero
    base = s * ROWS_TILE
    n_full = ROWS_TILE // K
    rem = ROWS_TILE % K
    for j in range(n_full):
        pltpu.sync_copy(rows_v.at[0], acc.at[pl.ds(base + j * K, K)])
    if rem:
        pltpu.sync_copy(rows_v.at[0, pl.ds(0, rem)],
                        acc.at[pl.ds(base + n_full * K, rem)])

    @pl.when(s == 0)
    def _zero_dummy():
        pltpu.sync_copy(rows_v.at[0, pl.ds(0, ACC_ROWS - ROWS_SC)],
                        acc.at[pl.ds(ROWS_SC, ACC_ROWS - ROWS_SC)])

    plsc.subcore_barrier()

    row_off = c * ROWS_SC
    gb0 = s * BLOCKS

    def idx_copy(b, sl):
        return pltpu.make_async_copy(
            packed_hbm.at[gb0 + b], idxw_v.at[sl], isem.at[sl])

    def w_copy(b, sl):
        return pltpu.make_async_copy(
            w_hbm.at[pl.ds((gb0 + b) * K, K)], w_v.at[sl], isem.at[sl])

    def gather_copy(b, sl):
        del b
        return pltpu.make_async_copy(
            x_hbm.at[idxw_v.at[sl, 0]], rows_v.at[sl], gsem.at[sl])

    def scatter_copy(sl):
        return pltpu.make_async_copy(
            rows_v.at[sl], acc.at[ldst_v.at[sl]], ssem.at[sl])

    def compute(sl):
        for g in range(K // 16):
            egrp = pl.ds(g * 16, 16)
            d16 = idxw_v[sl, 1, egrp]
            loc = d16 - row_off
            ok = (loc >= 0) & (loc < ROWS_SC)
            ldst_v[sl, egrp] = jnp.where(ok, loc, DUMMY)
            w16 = w_v[sl, egrp]
            for e in range(16):
                idx = g * 16 + e
                ws = w16[e]
                for d4 in range(D // 16):
                    dsl = pl.ds(d4 * 16, 16)
                    rows_v[sl, idx, dsl] = rows_v[sl, idx, dsl] * ws

    def block_step(b, sl, nsl, *, first_gather_wave, issue_next_idx,
                   issue_next_gather):
        if issue_next_gather:
            if not first_gather_wave:
                scatter_copy(nsl).wait()
            idx_copy(b + 1, nsl).wait()
            w_copy(b + 1, nsl).wait()
            gather_copy(b + 1, nsl).start()
        gather_copy(b, sl).wait()
        compute(sl)
        if issue_next_idx:
            idx_copy(b + NBUF, sl).start()
            w_copy(b + NBUF, sl).start()
        scatter_copy(sl).start(add=True)

    for sl in range(NBUF):
        idx_copy(sl, sl).start()
        w_copy(sl, sl).start()
    idx_copy(0, 0).wait()
    w_copy(0, 0).wait()
    gather_copy(0, 0).start()
    for b in range(NBUF):
        block_step(b, b, (b + 1) % NBUF,
                   first_gather_wave=(b < NBUF - 1),
                   issue_next_idx=True,
                   issue_next_gather=True)

    def body(t, carry):
        b0 = t * NBUF
        for u in range(NBUF):
            block_step(b0 + u, u, (u + 1) % NBUF,
                       first_gather_wave=False,
                       issue_next_idx=True,
                       issue_next_gather=True)
        return carry

    lax.fori_loop(1, BLOCKS // NBUF - 1, body, 0)

    for u in range(NBUF):
        b = BLOCKS - NBUF + u
        block_step(b, u, (u + 1) % NBUF,
                   first_gather_wave=False,
                   issue_next_idx=False,
                   issue_next_gather=(u < NBUF - 1))
    for sl in range(NBUF):
        scatter_copy(sl).wait()

    plsc.subcore_barrier()

    for j in range(n_full):
        pltpu.sync_copy(acc.at[pl.ds(base + j * K, K)], rows_v.at[0])
        pltpu.sync_copy(rows_v.at[0],
                        out_hbm.at[pl.ds(row_off + base + j * K, K)])
    if rem:
        pltpu.sync_copy(acc.at[pl.ds(base + n_full * K, rem)],
                        rows_v.at[0, pl.ds(0, rem)])
        pltpu.sync_copy(rows_v.at[0, pl.ds(0, rem)],
                        out_hbm.at[pl.ds(row_off + base + n_full * K, rem)])


def _mean_body(a_ref, b_ref, c_ref, d_ref, o_ref):
    o_ref[...] = (a_ref[...] + b_ref[...] + c_ref[...] + d_ref[...]) * 0.25


_BLK = 1000
_mean4 = pl.pallas_call(
    _mean_body,
    out_shape=jax.ShapeDtypeStruct((N_NODES, D), jnp.float32),
    grid=(N_NODES // _BLK,),
    in_specs=[pl.BlockSpec((_BLK, D), lambda i: (i, 0))] * 4,
    out_specs=pl.BlockSpec((_BLK, D), lambda i: (i, 0)),
)


def kernel(ItemAndUserEmebddings, edge_index, edge_weight):
    x = ItemAndUserEmebddings
    src = edge_index[1].astype(jnp.int32)
    dst = edge_index[0].astype(jnp.int32)
    pad = E_PAD - N_EDGES
    src_p = jnp.concatenate([src, jnp.zeros((pad,), jnp.int32)])
    dst_p = jnp.concatenate([dst, jnp.full((pad,), OUT_OF_RANGE, jnp.int32)])
    w_p = jnp.concatenate([edge_weight, jnp.zeros((pad,), jnp.float32)])
    packed = jnp.stack(
        [src_p.reshape(NS * BLOCKS, K),
         dst_p.reshape(NS * BLOCKS, K)], axis=1)
    x_p = jnp.concatenate(
        [x, jnp.zeros((N_PAD - N_NODES, D), jnp.float32)], axis=0)

    outs = [x]
    h = x_p
    for _ in range(LAYERS):
        h = _layer(h, packed, w_p)
        outs.append(h[:N_NODES])

    stacked = jnp.stack(outs, axis=1)
    mean = _mean4(outs[0], outs[1], outs[2], outs[3])
    return (mean, stacked)
